# baseline (speedup 1.0000x reference)
import functools

import jax
import jax.numpy as jnp
from jax import lax
from jax.experimental import pallas as pl
from jax.experimental.pallas import tpu as pltpu
from jax.experimental.pallas import tpu_sc as plsc

B, N, C = 64, 243, 512
HEADS = 8
HD = C // HEADS
CLUSTER = 81
KNN = 7
NPAD = 256
KPAD = 96
RSEL = 128
SQRT_C = float(C) ** 0.5
NEG_INF = float("-inf")
def _rowize(v, iota_r, iota_c):
    sel = iota_r == iota_c
    return jnp.sum(jnp.where(sel, jnp.broadcast_to(v, (NPAD, NPAD)), 0.0),
                   axis=0, keepdims=True)


def _colize(v, iota_r, iota_c):
    sel = iota_r == iota_c
    return jnp.sum(jnp.where(sel, jnp.broadcast_to(v, (NPAD, NPAD)), 0.0),
                   axis=1, keepdims=True)


def _score_body(x_ref, idx_ref):
    xb = x_ref[0]
    iota_r = lax.broadcasted_iota(jnp.int32, (NPAD, NPAD), 0)
    iota_c = lax.broadcasted_iota(jnp.int32, (NPAD, NPAD), 1)
    rowvalid = iota_r < N
    inf = jnp.float32(jnp.inf)

    n2 = jnp.sum(xb * xb, axis=1, keepdims=True)
    n2r = _rowize(n2, iota_r, iota_c)
    g = lax.dot_general(xb, xb, (((1,), (1,)), ((), ())),
                        preferred_element_type=jnp.float32)
    d2 = n2 + n2r - 2.0 * g
    d2c = jnp.maximum(d2, 0.0)

    work = jnp.where(rowvalid, d2c, inf)
    acc = jnp.zeros((1, NPAD), jnp.float32)
    left = jnp.full((1, NPAD), KNN, jnp.int32)
    for _ in range(KNN):
        m = jnp.min(work, axis=0, keepdims=True)
        hit = work == m
        cnt = jnp.sum(hit.astype(jnp.int32), axis=0, keepdims=True)
        take = jnp.minimum(cnt, left)
        left = left - take
        dnear = jnp.sqrt(m) / SQRT_C
        acc = acc + take.astype(jnp.float32) * (dnear * dnear)
        work = jnp.where(hit, inf, work)
    densr = jnp.exp(-(acc / KNN))
    densr = densr + lax.broadcasted_iota(
        jnp.int32, (1, NPAD), 1).astype(jnp.float32) * jnp.float32(1e-6)
    densc = _colize(densr, iota_r, iota_c)

    colvalid = iota_c < N
    d2max = jnp.max(jnp.where(rowvalid & colvalid, d2c, -inf))
    mask = (densc > densr) & rowvalid
    parent_d2 = jnp.min(jnp.where(mask, d2c, d2max), axis=0, keepdims=True)
    parent = jnp.sqrt(parent_d2) / SQRT_C

    scorer = parent * densr
    scorer = jnp.where(lax.broadcasted_iota(jnp.int32, (1, NPAD), 1) < N,
                       scorer, -inf)
    score = _colize(scorer, iota_r, iota_c)

    beat = (scorer > score) | ((scorer == score) & (iota_c < iota_r))
    rank = jnp.sum(beat.astype(jnp.int32), axis=1, keepdims=True)

    iota_rr = lax.broadcasted_iota(jnp.int32, (NPAD, RSEL), 1)
    ival = lax.broadcasted_iota(jnp.int32, (NPAD, RSEL), 0)
    out = jnp.sum(jnp.where(rank == iota_rr, ival, 0), axis=0, keepdims=True)
    idx_ref[...] = out.reshape(1, 1, RSEL)


def _score_call(x):
    return pl.pallas_call(
        _score_body,
        grid=(B,),
        in_specs=[pl.BlockSpec((1, NPAD, C), lambda b: (b, 0, 0))],
        out_specs=pl.BlockSpec((1, 1, RSEL), lambda b: (b, 0, 0)),
        out_shape=jax.ShapeDtypeStruct((B, 1, RSEL), jnp.int32),
    )(x)


_NC = 2
_NS = 16
_NW = _NC * _NS
_B_PER_W = B // _NW
_PER_W = _B_PER_W * KPAD


def _gather_body(x_hbm, idx_hbm, out_hbm, idx_v, rows_v, sem):
    wid = lax.axis_index("s") * _NC + lax.axis_index("c")
    b0 = wid * _B_PER_W
    pltpu.sync_copy(idx_hbm.at[pl.ds(b0, _B_PER_W)], idx_v)
    cps = [
        pltpu.async_copy(x_hbm.at[b0 + j].at[idx_v.at[j]],
                         rows_v.at[pl.ds(j * KPAD, KPAD)], sem)
        for j in range(_B_PER_W)
    ]
    for cp in cps:
        cp.wait()
    pltpu.sync_copy(rows_v, out_hbm.at[pl.ds(b0 * KPAD, _PER_W)])


@functools.cache
def _gather_call():
    return pl.kernel(
        _gather_body,
        out_type=jax.ShapeDtypeStruct((B * KPAD, C), jnp.float32),
        mesh=plsc.VectorSubcoreMesh(core_axis_name="c", subcore_axis_name="s"),
        scratch_types=[
            pltpu.VMEM((_B_PER_W, KPAD), jnp.int32),
            pltpu.VMEM((_PER_W, C), jnp.float32),
            pltpu.SemaphoreType.DMA,
        ],
    )


BB = 2


def _attn_body(x_ref, cen_ref, pos_ref, wq_ref, wk_ref, wv_ref, wp_ref,
               bp_ref, o_ref):
    pos = pos_ref[0]
    scale = jnp.float32(HD ** -0.5)
    kmask = lax.broadcasted_iota(jnp.int32, (NPAD, KPAD), 1) < CLUSTER
    for t in range(BB):
        xb = x_ref[t]
        cen = cen_ref[t * KPAD:(t + 1) * KPAD, :] + pos
        xb16 = xb.astype(jnp.bfloat16)
        cen16 = cen.astype(jnp.bfloat16)
        q = jnp.dot(xb16, wq_ref[...], preferred_element_type=jnp.float32)
        k = jnp.dot(cen16, wk_ref[...], preferred_element_type=jnp.float32)
        v = jnp.dot(cen16, wv_ref[...],
                    preferred_element_type=jnp.float32).astype(jnp.bfloat16)
        outs = []
        for h in range(HEADS):
            qh = q[:, h * HD:(h + 1) * HD].astype(jnp.bfloat16)
            kh = k[:, h * HD:(h + 1) * HD].astype(jnp.bfloat16)
            vh = v[:, h * HD:(h + 1) * HD]
            s = lax.dot_general(qh, kh, (((1,), (1,)), ((), ())),
                                preferred_element_type=jnp.float32) * scale
            e = jnp.where(kmask, jnp.exp(s), 0.0)
            r = 1.0 / jnp.sum(e, axis=1, keepdims=True)
            av = jnp.dot(e.astype(jnp.bfloat16), vh,
                         preferred_element_type=jnp.float32)
            outs.append(av * r)
        o = jnp.concatenate(outs, axis=1).astype(jnp.bfloat16)
        res = jnp.dot(o, wp_ref[...], preferred_element_type=jnp.float32)
        o_ref[t] = res + bp_ref[...] + xb


def _attn_call(x, centers_flat, pos_pad, Wq, Wk, Wv, Wp, bp2):
    wspec = pl.BlockSpec((C, C), lambda b: (0, 0))
    return pl.pallas_call(
        _attn_body,
        grid=(B // BB,),
        in_specs=[
            pl.BlockSpec((BB, NPAD, C), lambda b: (b, 0, 0)),
            pl.BlockSpec((BB * KPAD, C), lambda b: (b, 0)),
            pl.BlockSpec((1, KPAD, C), lambda b: (0, 0, 0)),
            wspec, wspec, wspec, wspec,
            pl.BlockSpec((1, C), lambda b: (0, 0)),
        ],
        out_specs=pl.BlockSpec((BB, NPAD, C), lambda b: (b, 0, 0)),
        out_shape=jax.ShapeDtypeStruct((B, NPAD, C), jnp.float32),
    )(x, centers_flat, pos_pad, Wq, Wk, Wv, Wp, bp2)


def kernel(x, Wq, Wk, Wv, Wp, bp, pos_embed):
    idx3 = _score_call(x)
    idx2 = idx3[:, 0, :KPAD]
    centers_flat = _gather_call()(x, idx2)
    pos_pad = jnp.pad(pos_embed, ((0, 0), (0, KPAD - CLUSTER), (0, 0)))
    out = _attn_call(x, centers_flat, pos_pad,
                     Wq.astype(jnp.bfloat16), Wk.astype(jnp.bfloat16),
                     Wv.astype(jnp.bfloat16), Wp.astype(jnp.bfloat16),
                     bp.reshape(1, C))
    return out[:, :N, :]


# speedup vs baseline: 1.1614x; 1.1614x over previous
import functools

import jax
import jax.numpy as jnp
from jax import lax
from jax.experimental import pallas as pl
from jax.experimental.pallas import tpu as pltpu
from jax.experimental.pallas import tpu_sc as plsc

B, N, C = 64, 243, 512
HEADS = 8
HD = C // HEADS
CLUSTER = 81
KNN = 7
NPAD = 256
KPAD = 96
RSEL = 128
SQRT_C = float(C) ** 0.5
NEG_INF = float("-inf")
def _rowize(v, iota_r, iota_c):
    sel = iota_r == iota_c
    return jnp.sum(jnp.where(sel, jnp.broadcast_to(v, (NPAD, NPAD)), 0.0),
                   axis=0, keepdims=True)


def _colize(v, iota_r, iota_c):
    sel = iota_r == iota_c
    return jnp.sum(jnp.where(sel, jnp.broadcast_to(v, (NPAD, NPAD)), 0.0),
                   axis=1, keepdims=True)


def _score_body(x_ref, idx_ref):
    for t in range(BB):
        _score_one(x_ref[t], idx_ref, t)


def _score_one(xb, idx_ref, t):
    iota_r = lax.broadcasted_iota(jnp.int32, (NPAD, NPAD), 0)
    iota_c = lax.broadcasted_iota(jnp.int32, (NPAD, NPAD), 1)
    rowvalid = iota_r < N
    inf = jnp.float32(jnp.inf)

    n2 = jnp.sum(xb * xb, axis=1, keepdims=True)
    n2r = _rowize(n2, iota_r, iota_c)
    g = lax.dot_general(xb, xb, (((1,), (1,)), ((), ())),
                        preferred_element_type=jnp.float32)
    d2 = n2 + n2r - 2.0 * g
    d2c = jnp.maximum(d2, 0.0)

    work = jnp.where(rowvalid, d2c, inf)
    acc = jnp.zeros((1, NPAD), jnp.float32)
    left = jnp.full((1, NPAD), KNN, jnp.int32)
    for _ in range(KNN):
        m = jnp.min(work, axis=0, keepdims=True)
        hit = work == m
        cnt = jnp.sum(hit.astype(jnp.int32), axis=0, keepdims=True)
        take = jnp.minimum(cnt, left)
        left = left - take
        dnear = jnp.sqrt(m) / SQRT_C
        acc = acc + take.astype(jnp.float32) * (dnear * dnear)
        work = jnp.where(hit, inf, work)
    densr = jnp.exp(-(acc / KNN))
    densr = densr + lax.broadcasted_iota(
        jnp.int32, (1, NPAD), 1).astype(jnp.float32) * jnp.float32(1e-6)
    densc = _colize(densr, iota_r, iota_c)

    colvalid = iota_c < N
    d2max = jnp.max(jnp.where(rowvalid & colvalid, d2c, -inf))
    mask = (densc > densr) & rowvalid
    parent_d2 = jnp.min(jnp.where(mask, d2c, d2max), axis=0, keepdims=True)
    parent = jnp.sqrt(parent_d2) / SQRT_C

    scorer = parent * densr
    scorer = jnp.where(lax.broadcasted_iota(jnp.int32, (1, NPAD), 1) < N,
                       scorer, -inf)
    score = _colize(scorer, iota_r, iota_c)

    beat = (scorer > score) | ((scorer == score) & (iota_c < iota_r))
    rank = jnp.sum(beat.astype(jnp.int32), axis=1, keepdims=True)

    iota_rr = lax.broadcasted_iota(jnp.int32, (NPAD, RSEL), 1)
    ival = lax.broadcasted_iota(jnp.int32, (NPAD, RSEL), 0)
    out = jnp.sum(jnp.where(rank == iota_rr, ival, 0), axis=0, keepdims=True)
    idx_ref[t] = out.reshape(1, RSEL)


def _score_call(x):
    return pl.pallas_call(
        _score_body,
        grid=(B // BB,),
        in_specs=[pl.BlockSpec((BB, NPAD, C), lambda b: (b, 0, 0))],
        out_specs=pl.BlockSpec((BB, 1, RSEL), lambda b: (b, 0, 0)),
        out_shape=jax.ShapeDtypeStruct((B, 1, RSEL), jnp.int32),
    )(x)


_NC = 2
_NS = 16
_NW = _NC * _NS
_B_PER_W = B // _NW
_PER_W = _B_PER_W * KPAD


def _gather_body(x_hbm, idx_hbm, out_hbm, idx_v, rows_v, sem):
    wid = lax.axis_index("s") * _NC + lax.axis_index("c")
    b0 = wid * _B_PER_W
    pltpu.sync_copy(idx_hbm.at[pl.ds(b0, _B_PER_W)], idx_v)
    cps = [
        pltpu.async_copy(x_hbm.at[b0 + j].at[idx_v.at[j]],
                         rows_v.at[pl.ds(j * KPAD, KPAD)], sem)
        for j in range(_B_PER_W)
    ]
    for cp in cps:
        cp.wait()
    pltpu.sync_copy(rows_v, out_hbm.at[pl.ds(b0 * KPAD, _PER_W)])


@functools.cache
def _gather_call():
    return pl.kernel(
        _gather_body,
        out_type=jax.ShapeDtypeStruct((B * KPAD, C), jnp.float32),
        mesh=plsc.VectorSubcoreMesh(core_axis_name="c", subcore_axis_name="s"),
        scratch_types=[
            pltpu.VMEM((_B_PER_W, KPAD), jnp.int32),
            pltpu.VMEM((_PER_W, C), jnp.float32),
            pltpu.SemaphoreType.DMA,
        ],
    )


BB = 2


def _attn_body(x_ref, cen_ref, pos_ref, wq_ref, wk_ref, wv_ref, wp_ref,
               bp_ref, o_ref):
    pos = pos_ref[0]
    scale = jnp.float32(HD ** -0.5)
    kmask = lax.broadcasted_iota(jnp.int32, (NPAD, KPAD), 1) < CLUSTER
    for t in range(BB):
        xb = x_ref[t]
        cen = cen_ref[t * KPAD:(t + 1) * KPAD, :] + pos
        xb16 = xb.astype(jnp.bfloat16)
        cen16 = cen.astype(jnp.bfloat16)
        q = jnp.dot(xb16, wq_ref[...], preferred_element_type=jnp.float32)
        k = jnp.dot(cen16, wk_ref[...], preferred_element_type=jnp.float32)
        v = jnp.dot(cen16, wv_ref[...],
                    preferred_element_type=jnp.float32).astype(jnp.bfloat16)
        outs = []
        for h in range(HEADS):
            qh = q[:, h * HD:(h + 1) * HD].astype(jnp.bfloat16)
            kh = k[:, h * HD:(h + 1) * HD].astype(jnp.bfloat16)
            vh = v[:, h * HD:(h + 1) * HD]
            s = lax.dot_general(qh, kh, (((1,), (1,)), ((), ())),
                                preferred_element_type=jnp.float32) * scale
            e = jnp.where(kmask, jnp.exp(s), 0.0)
            r = 1.0 / jnp.sum(e, axis=1, keepdims=True)
            av = jnp.dot(e.astype(jnp.bfloat16), vh,
                         preferred_element_type=jnp.float32)
            outs.append(av * r)
        o = jnp.concatenate(outs, axis=1).astype(jnp.bfloat16)
        res = jnp.dot(o, wp_ref[...], preferred_element_type=jnp.float32)
        o_ref[t] = (res + bp_ref[...] + xb)[:N]


def _attn_call(x, centers_flat, pos_pad, Wq, Wk, Wv, Wp, bp2):
    wspec = pl.BlockSpec((C, C), lambda b: (0, 0))
    return pl.pallas_call(
        _attn_body,
        grid=(B // BB,),
        in_specs=[
            pl.BlockSpec((BB, NPAD, C), lambda b: (b, 0, 0)),
            pl.BlockSpec((BB * KPAD, C), lambda b: (b, 0)),
            pl.BlockSpec((1, KPAD, C), lambda b: (0, 0, 0)),
            wspec, wspec, wspec, wspec,
            pl.BlockSpec((1, C), lambda b: (0, 0)),
        ],
        out_specs=pl.BlockSpec((BB, N, C), lambda b: (b, 0, 0)),
        out_shape=jax.ShapeDtypeStruct((B, N, C), jnp.float32),
    )(x, centers_flat, pos_pad, Wq, Wk, Wv, Wp, bp2)


def kernel(x, Wq, Wk, Wv, Wp, bp, pos_embed):
    idx3 = _score_call(x)
    idx2 = idx3[:, 0, :KPAD]
    centers_flat = _gather_call()(x, idx2)
    pos_pad = jnp.pad(pos_embed, ((0, 0), (0, KPAD - CLUSTER), (0, 0)))
    return _attn_call(x, centers_flat, pos_pad,
                      Wq.astype(jnp.bfloat16), Wk.astype(jnp.bfloat16),
                      Wv.astype(jnp.bfloat16), Wp.astype(jnp.bfloat16),
                      bp.reshape(1, C))


# speedup vs baseline: 1.1729x; 1.0099x over previous
import functools

import jax
import jax.numpy as jnp
from jax import lax
from jax.experimental import pallas as pl
from jax.experimental.pallas import tpu as pltpu
from jax.experimental.pallas import tpu_sc as plsc

B, N, C = 64, 243, 512
HEADS = 8
HD = C // HEADS
CLUSTER = 81
KNN = 7
NPAD = 256
KPAD = 96
RSEL = 128
SQRT_C = float(C) ** 0.5
NEG_INF = float("-inf")
def _rowize(v, iota_r, iota_c):
    sel = iota_r == iota_c
    return jnp.sum(jnp.where(sel, jnp.broadcast_to(v, (NPAD, NPAD)), 0.0),
                   axis=0, keepdims=True)


def _colize(v, iota_r, iota_c):
    sel = iota_r == iota_c
    return jnp.sum(jnp.where(sel, jnp.broadcast_to(v, (NPAD, NPAD)), 0.0),
                   axis=1, keepdims=True)


def _score_body(x_ref, idx_ref):
    for t in range(BB):
        _score_one(x_ref[t], idx_ref, t)


def _score_one(xb, idx_ref, t):
    iota_r = lax.broadcasted_iota(jnp.int32, (NPAD, NPAD), 0)
    iota_c = lax.broadcasted_iota(jnp.int32, (NPAD, NPAD), 1)
    rowvalid = iota_r < N
    inf = jnp.float32(jnp.inf)

    n2 = jnp.sum(xb * xb, axis=1, keepdims=True)
    n2r = _rowize(n2, iota_r, iota_c)
    g = lax.dot_general(xb, xb, (((1,), (1,)), ((), ())),
                        preferred_element_type=jnp.float32)
    d2 = n2 + n2r - 2.0 * g
    d2c = jnp.maximum(d2, 0.0)

    work = jnp.where(rowvalid, d2c, inf)
    acc = jnp.zeros((1, NPAD), jnp.float32)
    left = jnp.full((1, NPAD), KNN, jnp.int32)
    for _ in range(KNN):
        m = jnp.min(work, axis=0, keepdims=True)
        hit = work == m
        cnt = jnp.sum(hit.astype(jnp.int32), axis=0, keepdims=True)
        take = jnp.minimum(cnt, left)
        left = left - take
        dnear = jnp.sqrt(m) / SQRT_C
        acc = acc + take.astype(jnp.float32) * (dnear * dnear)
        work = jnp.where(hit, inf, work)
    densr = jnp.exp(-(acc / KNN))
    densr = densr + lax.broadcasted_iota(
        jnp.int32, (1, NPAD), 1).astype(jnp.float32) * jnp.float32(1e-6)
    densc = _colize(densr, iota_r, iota_c)

    colvalid = iota_c < N
    d2max = jnp.max(jnp.where(rowvalid & colvalid, d2c, -inf))
    mask = (densc > densr) & rowvalid
    parent_d2 = jnp.min(jnp.where(mask, d2c, d2max), axis=0, keepdims=True)
    parent = jnp.sqrt(parent_d2) / SQRT_C

    scorer = parent * densr
    scorer = jnp.where(lax.broadcasted_iota(jnp.int32, (1, NPAD), 1) < N,
                       scorer, -inf)
    score = _colize(scorer, iota_r, iota_c)

    beat = (scorer > score) | ((scorer == score) & (iota_c < iota_r))
    rank = jnp.sum(beat.astype(jnp.int32), axis=1, keepdims=True)

    iota_rr = lax.broadcasted_iota(jnp.int32, (NPAD, RSEL), 1)
    ival = lax.broadcasted_iota(jnp.int32, (NPAD, RSEL), 0)
    out = jnp.sum(jnp.where(rank == iota_rr, ival, 0), axis=0, keepdims=True)
    idx_ref[t] = out.reshape(1, RSEL)


def _score_call(x):
    return pl.pallas_call(
        _score_body,
        grid=(B // BB,),
        in_specs=[pl.BlockSpec((BB, NPAD, C), lambda b: (b, 0, 0))],
        out_specs=pl.BlockSpec((BB, 1, RSEL), lambda b: (b, 0, 0)),
        out_shape=jax.ShapeDtypeStruct((B, 1, RSEL), jnp.int32),
    )(x)


_NC = 2
_NS = 16
_NW = _NC * _NS
_B_PER_W = B // _NW
_PER_W = _B_PER_W * KPAD


def _gather_body(x_hbm, idx_hbm, out_hbm, idx_v, rows_v, sem):
    wid = lax.axis_index("s") * _NC + lax.axis_index("c")
    b0 = wid * _B_PER_W
    pltpu.sync_copy(idx_hbm.at[pl.ds(b0, _B_PER_W)], idx_v)
    cps = [
        pltpu.async_copy(x_hbm.at[b0 + j].at[idx_v.at[j]],
                         rows_v.at[pl.ds(j * KPAD, KPAD)], sem)
        for j in range(_B_PER_W)
    ]
    for cp in cps:
        cp.wait()
    pltpu.sync_copy(rows_v, out_hbm.at[pl.ds(b0 * KPAD, _PER_W)])


@functools.cache
def _gather_call():
    return pl.kernel(
        _gather_body,
        out_type=jax.ShapeDtypeStruct((B * KPAD, C), jnp.float32),
        mesh=plsc.VectorSubcoreMesh(core_axis_name="c", subcore_axis_name="s"),
        scratch_types=[
            pltpu.VMEM((_B_PER_W, KPAD), jnp.int32),
            pltpu.VMEM((_PER_W, C), jnp.float32),
            pltpu.SemaphoreType.DMA,
        ],
    )


BB = 4


def _attn_body(x_ref, cen_ref, pos_ref, wq_ref, wk_ref, wv_ref, wp_ref,
               bp_ref, o_ref):
    pos = pos_ref[0]
    scale = jnp.float32(HD ** -0.5)
    kmask = lax.broadcasted_iota(jnp.int32, (NPAD, KPAD), 1) < CLUSTER
    for t in range(BB):
        xb = x_ref[t]
        cen = cen_ref[t * KPAD:(t + 1) * KPAD, :] + pos
        xb16 = xb.astype(jnp.bfloat16)
        cen16 = cen.astype(jnp.bfloat16)
        q = jnp.dot(xb16, wq_ref[...], preferred_element_type=jnp.float32)
        k = jnp.dot(cen16, wk_ref[...], preferred_element_type=jnp.float32)
        v = jnp.dot(cen16, wv_ref[...],
                    preferred_element_type=jnp.float32).astype(jnp.bfloat16)
        outs = []
        for h in range(HEADS):
            qh = q[:, h * HD:(h + 1) * HD].astype(jnp.bfloat16)
            kh = k[:, h * HD:(h + 1) * HD].astype(jnp.bfloat16)
            vh = v[:, h * HD:(h + 1) * HD]
            s = lax.dot_general(qh, kh, (((1,), (1,)), ((), ())),
                                preferred_element_type=jnp.float32) * scale
            e = jnp.where(kmask, jnp.exp(s), 0.0)
            r = 1.0 / jnp.sum(e, axis=1, keepdims=True)
            av = jnp.dot(e.astype(jnp.bfloat16), vh,
                         preferred_element_type=jnp.float32)
            outs.append(av * r)
        o = jnp.concatenate(outs, axis=1).astype(jnp.bfloat16)
        res = jnp.dot(o, wp_ref[...], preferred_element_type=jnp.float32)
        o_ref[t] = (res + bp_ref[...] + xb)[:N]


def _attn_call(x, centers_flat, pos_pad, Wq, Wk, Wv, Wp, bp2):
    wspec = pl.BlockSpec((C, C), lambda b: (0, 0))
    return pl.pallas_call(
        _attn_body,
        grid=(B // BB,),
        in_specs=[
            pl.BlockSpec((BB, NPAD, C), lambda b: (b, 0, 0)),
            pl.BlockSpec((BB * KPAD, C), lambda b: (b, 0)),
            pl.BlockSpec((1, KPAD, C), lambda b: (0, 0, 0)),
            wspec, wspec, wspec, wspec,
            pl.BlockSpec((1, C), lambda b: (0, 0)),
        ],
        out_specs=pl.BlockSpec((BB, N, C), lambda b: (b, 0, 0)),
        out_shape=jax.ShapeDtypeStruct((B, N, C), jnp.float32),
    )(x, centers_flat, pos_pad, Wq, Wk, Wv, Wp, bp2)


def kernel(x, Wq, Wk, Wv, Wp, bp, pos_embed):
    idx3 = _score_call(x)
    idx2 = idx3[:, 0, :KPAD]
    centers_flat = _gather_call()(x, idx2)
    pos_pad = jnp.pad(pos_embed, ((0, 0), (0, KPAD - CLUSTER), (0, 0)))
    return _attn_call(x, centers_flat, pos_pad,
                      Wq.astype(jnp.bfloat16), Wk.astype(jnp.bfloat16),
                      Wv.astype(jnp.bfloat16), Wp.astype(jnp.bfloat16),
                      bp.reshape(1, C))
